# baseline (speedup 1.0000x reference)
import functools

import jax
import jax.numpy as jnp
from jax import lax
from jax.experimental import pallas as pl
from jax.experimental.pallas import tpu as pltpu
from jax.experimental.pallas import tpu_sc as plsc

_CHUNKS = 16
_NBUF = 6
_LOOKAHEAD = 3


def _sc_busy_probe(B, L, iters):
    mesh = plsc.VectorSubcoreMesh(core_axis_name="c", subcore_axis_name="s")

    @functools.partial(
        pl.kernel,
        mesh=mesh,
        out_type=jax.ShapeDtypeStruct((B + 1,), jnp.int32),
        scratch_types=[pltpu.VMEM((32,), jnp.int32)],
    )
    def k(off_hbm, off_v):
        cid = lax.axis_index("c")
        sid = lax.axis_index("s")

        @pl.when(jnp.logical_and(cid == 0, sid == 0))
        def _():
            lane = lax.iota(jnp.int32, 16)

            def body(_, acc):
                return acc + 1

            spin = lax.fori_loop(0, iters, body, jnp.zeros((16,), jnp.int32))
            off_v[pl.ds(0, 16)] = lane * L + (spin - iters)
            off_v[pl.ds(16, 16)] = (lane + 16) * L
            pltpu.sync_copy(off_v.at[pl.ds(0, B + 1)], off_hbm)

    return k()


def _body(x_hbm, o_hbm, off_hbm, buf, off_smem, in_sems, out_sems, off_sem):
    n_rows = x_hbm.shape[0]
    b = off_hbm.shape[0] - 1
    seq_len = n_rows // b
    for i in range(b + 1):
        off_smem[i] = i * seq_len
    off_copy = pltpu.make_async_copy(off_smem, off_hbm, off_sem)
    off_copy.start()

    rows = n_rows // _CHUNKS
    ins = [
        pltpu.make_async_copy(
            x_hbm.at[pl.ds(k * rows, rows)], buf.at[k % _NBUF],
            in_sems.at[k % _NBUF],
        )
        for k in range(_CHUNKS)
    ]
    outs = [
        pltpu.make_async_copy(
            buf.at[k % _NBUF], o_hbm.at[pl.ds(k * rows, rows)],
            out_sems.at[k % _NBUF],
        )
        for k in range(_CHUNKS)
    ]
    for k in range(_LOOKAHEAD):
        ins[k].start()
    for k in range(_CHUNKS):
        if k >= _LOOKAHEAD:
            outs[k - _LOOKAHEAD].wait()
        if k + _LOOKAHEAD < _CHUNKS:
            ins[k + _LOOKAHEAD].start()
        ins[k].wait()
        outs[k].start()
    for k in range(_CHUNKS - _LOOKAHEAD, _CHUNKS):
        outs[k].wait()
    off_copy.wait()


def kernel(X):
    B, L, d = X.shape
    x_flat = X.reshape(B * L, d)
    n_rows = B * L
    rows = n_rows // _CHUNKS
    offsets_sc = _sc_busy_probe(B, L, 500000)
    values, _ = pl.pallas_call(
        _body,
        in_specs=[pl.BlockSpec(memory_space=pl.ANY)],
        out_specs=[
            pl.BlockSpec(memory_space=pl.ANY),
            pl.BlockSpec(memory_space=pl.ANY),
        ],
        out_shape=[
            jax.ShapeDtypeStruct((n_rows, d), x_flat.dtype),
            jax.ShapeDtypeStruct((B + 1,), jnp.int32),
        ],
        scratch_shapes=[
            pltpu.VMEM((_NBUF, rows, d), x_flat.dtype),
            pltpu.SMEM((B + 1,), jnp.int32),
            pltpu.SemaphoreType.DMA((_NBUF,)),
            pltpu.SemaphoreType.DMA((_NBUF,)),
            pltpu.SemaphoreType.DMA,
        ],
    )(x_flat)
    return (values, offsets_sc)


# speedup vs baseline: 24.2772x; 24.2772x over previous
import jax
import jax.numpy as jnp
from jax.experimental import pallas as pl
from jax.experimental.pallas import tpu as pltpu

_CHUNKS = 16
_NBUF = 6
_LOOKAHEAD = 3


def _body(x_hbm, o_hbm, off_hbm, buf, off_smem, in_sems, out_sems, off_sem):
    n_rows = x_hbm.shape[0]
    b = off_hbm.shape[0] - 1
    seq_len = n_rows // b
    for i in range(b + 1):
        off_smem[i] = i * seq_len
    off_copy = pltpu.make_async_copy(off_smem, off_hbm, off_sem)
    off_copy.start()

    rows = n_rows // _CHUNKS
    ins = [
        pltpu.make_async_copy(
            x_hbm.at[pl.ds(k * rows, rows)], buf.at[k % _NBUF],
            in_sems.at[k % _NBUF],
        )
        for k in range(_CHUNKS)
    ]
    outs = [
        pltpu.make_async_copy(
            buf.at[k % _NBUF], o_hbm.at[pl.ds(k * rows, rows)],
            out_sems.at[k % _NBUF],
        )
        for k in range(_CHUNKS)
    ]
    for k in range(_LOOKAHEAD):
        ins[k].start()
    for k in range(_CHUNKS):
        if k >= _LOOKAHEAD:
            outs[k - _LOOKAHEAD].wait()
        if k + _LOOKAHEAD < _CHUNKS:
            ins[k + _LOOKAHEAD].start()
        ins[k].wait()
        outs[k].start()
    for k in range(_CHUNKS - _LOOKAHEAD, _CHUNKS):
        outs[k].wait()
    off_copy.wait()


def kernel(X):
    B, L, d = X.shape
    x_flat = X.reshape(B * L, d)
    n_rows = B * L
    rows = n_rows // _CHUNKS
    values, offsets = pl.pallas_call(
        _body,
        in_specs=[pl.BlockSpec(memory_space=pl.ANY)],
        out_specs=[
            pl.BlockSpec(memory_space=pl.ANY),
            pl.BlockSpec(memory_space=pl.ANY),
        ],
        out_shape=[
            jax.ShapeDtypeStruct((n_rows, d), x_flat.dtype),
            jax.ShapeDtypeStruct((B + 1,), jnp.int32),
        ],
        scratch_shapes=[
            pltpu.VMEM((_NBUF, rows, d), x_flat.dtype),
            pltpu.SMEM((B + 1,), jnp.int32),
            pltpu.SemaphoreType.DMA((_NBUF,)),
            pltpu.SemaphoreType.DMA((_NBUF,)),
            pltpu.SemaphoreType.DMA,
        ],
    )(x_flat)
    return (values, offsets)
